# baseline (speedup 1.0000x reference)
import jax
import jax.numpy as jnp
from jax import lax
from jax.experimental import pallas as pl

_MIN_TEMPO = 90.0
_MAX_TEMPO = 140.0
_KCAT = 128


def _prep_body(kt_ref, st_ref, w2_ref, b2_ref, wp_ref, bp_ref, vcat_ref):
    wp_key = wp_ref[0:64, :]
    wp_tmp = wp_ref[64:128, :]
    wp_sig = wp_ref[128:192, :]
    ktf = jnp.dot(kt_ref[...], wp_key, preferred_element_type=jnp.float32)
    stf = jnp.dot(st_ref[...], wp_sig, preferred_element_type=jnp.float32)
    w2f = jnp.dot(w2_ref[...], wp_tmp, preferred_element_type=jnp.float32)
    crow = jnp.dot(b2_ref[...], wp_tmp,
                   preferred_element_type=jnp.float32) + bp_ref[...]
    pad = jnp.zeros((_KCAT - 69, vcat_ref.shape[1]), jnp.float32)
    vcat_ref[...] = jnp.concatenate([ktf, stf, w2f, crow, pad], axis=0)


def _main_body(kid_ref, tv_ref, sid_ref, w1p_ref, b1p_ref, vcat_ref, out_ref):
    blk = out_ref.shape[0]
    kid = kid_ref[0, 0, :]
    sid = sid_ref[0, 0, :]
    tv = tv_ref[0, 0, :]

    lane = lax.broadcasted_iota(jnp.int32, (blk, _KCAT), 1)
    ohk = (kid[:, None] == lane).astype(jnp.float32)
    ohs = ((sid[:, None] + 26) == lane).astype(jnp.float32)
    tn = jnp.where(tv > 0, (tv - _MIN_TEMPO) / (_MAX_TEMPO - _MIN_TEMPO),
                   jnp.zeros_like(tv))
    h = jnp.maximum(tn[:, None] * w1p_ref[...] + b1p_ref[...], 0.0)
    m = ohk + ohs + h
    out_ref[...] = jnp.dot(m, vcat_ref[...],
                           preferred_element_type=jnp.float32)


def kernel(key_ids, tempo_values, time_sig_ids, key_table, time_sig_table,
           W1, b1, W2, b2, Wp, bp):
    B = key_ids.shape[0]
    H = Wp.shape[1]
    T = W1.shape[1]
    BLK = 2048 if B % 2048 == 0 else B
    NB = B // BLK

    def full_spec(shape):
        nd = len(shape)
        return pl.BlockSpec(shape, lambda i, _nd=nd: (0,) * _nd)

    b2r = b2.reshape(1, -1)
    bpr = bp.reshape(1, -1)
    vcat = pl.pallas_call(
        _prep_body,
        out_shape=jax.ShapeDtypeStruct((_KCAT, H), jnp.float32),
    )(key_table, time_sig_table, W2, b2r, Wp, bpr)

    w1pad = jnp.zeros((1, _KCAT), jnp.float32).at[0, 36:36 + T].set(W1[0])
    b1pad = (jnp.zeros((1, _KCAT), jnp.float32)
             .at[0, 36:36 + T].set(b1).at[0, 68].set(1.0))

    kid3 = key_ids.astype(jnp.int32).reshape(NB, 1, BLK)
    tv3 = tempo_values.reshape(NB, 1, BLK)
    sid3 = time_sig_ids.astype(jnp.int32).reshape(NB, 1, BLK)

    return pl.pallas_call(
        _main_body,
        grid=(NB,),
        in_specs=[
            pl.BlockSpec((1, 1, BLK), lambda i: (i, 0, 0)),
            pl.BlockSpec((1, 1, BLK), lambda i: (i, 0, 0)),
            pl.BlockSpec((1, 1, BLK), lambda i: (i, 0, 0)),
            full_spec((1, _KCAT)),
            full_spec((1, _KCAT)),
            full_spec((_KCAT, H)),
        ],
        out_specs=pl.BlockSpec((BLK, H), lambda i: (i, 0)),
        out_shape=jax.ShapeDtypeStruct((B, H), jnp.float32),
    )(kid3, tv3, sid3, w1pad, b1pad, vcat)


# speedup vs baseline: 1.0482x; 1.0482x over previous
import jax
import jax.numpy as jnp
from jax import lax
from jax.experimental import pallas as pl
from jax.experimental.pallas import tpu as pltpu

_MIN_TEMPO = 90.0
_MAX_TEMPO = 140.0
_KCAT = 128


def _body(kid_ref, tv_ref, sid_ref, w1p_ref, b1p_ref, kt_ref, st_ref,
          w2_ref, b2_ref, wp_ref, bp_ref, out_ref, vcat_ref):
    blk = out_ref.shape[0]

    @pl.when(pl.program_id(0) == 0)
    def _fold():
        wp_key = wp_ref[0:64, :]
        wp_tmp = wp_ref[64:128, :]
        wp_sig = wp_ref[128:192, :]
        ktf = jnp.dot(kt_ref[...], wp_key, preferred_element_type=jnp.float32)
        stf = jnp.dot(st_ref[...], wp_sig, preferred_element_type=jnp.float32)
        w2f = jnp.dot(w2_ref[...], wp_tmp, preferred_element_type=jnp.float32)
        crow = jnp.dot(b2_ref[...], wp_tmp,
                       preferred_element_type=jnp.float32) + bp_ref[...]
        pad = jnp.zeros((_KCAT - 69, out_ref.shape[1]), jnp.float32)
        vcat_ref[...] = jnp.concatenate([ktf, stf, w2f, crow, pad], axis=0)

    kid = kid_ref[0, 0, :]
    sid = sid_ref[0, 0, :]
    tv = tv_ref[0, 0, :]

    lane = lax.broadcasted_iota(jnp.int32, (blk, _KCAT), 1)
    ohk = (kid[:, None] == lane).astype(jnp.float32)
    ohs = ((sid[:, None] + 26) == lane).astype(jnp.float32)
    tn = jnp.where(tv > 0, (tv - _MIN_TEMPO) / (_MAX_TEMPO - _MIN_TEMPO),
                   jnp.zeros_like(tv))
    h = jnp.maximum(tn[:, None] * w1p_ref[...] + b1p_ref[...], 0.0)
    m = ohk + ohs + h
    out_ref[...] = jnp.dot(m, vcat_ref[...],
                           preferred_element_type=jnp.float32)


def kernel(key_ids, tempo_values, time_sig_ids, key_table, time_sig_table,
           W1, b1, W2, b2, Wp, bp):
    B = key_ids.shape[0]
    H = Wp.shape[1]
    T = W1.shape[1]
    BLK = 2048 if B % 2048 == 0 else B
    NB = B // BLK

    def full_spec(shape):
        nd = len(shape)
        return pl.BlockSpec(shape, lambda i, _nd=nd: (0,) * _nd)

    b2r = b2.reshape(1, -1)
    bpr = bp.reshape(1, -1)
    w1pad = jnp.zeros((1, _KCAT), jnp.float32).at[0, 36:36 + T].set(W1[0])
    b1pad = (jnp.zeros((1, _KCAT), jnp.float32)
             .at[0, 36:36 + T].set(b1).at[0, 68].set(1.0))

    kid3 = key_ids.astype(jnp.int32).reshape(NB, 1, BLK)
    tv3 = tempo_values.reshape(NB, 1, BLK)
    sid3 = time_sig_ids.astype(jnp.int32).reshape(NB, 1, BLK)

    return pl.pallas_call(
        _body,
        grid=(NB,),
        in_specs=[
            pl.BlockSpec((1, 1, BLK), lambda i: (i, 0, 0)),
            pl.BlockSpec((1, 1, BLK), lambda i: (i, 0, 0)),
            pl.BlockSpec((1, 1, BLK), lambda i: (i, 0, 0)),
            full_spec((1, _KCAT)),
            full_spec((1, _KCAT)),
            full_spec(key_table.shape),
            full_spec(time_sig_table.shape),
            full_spec(W2.shape),
            full_spec(b2r.shape),
            full_spec(Wp.shape),
            full_spec(bpr.shape),
        ],
        out_specs=pl.BlockSpec((BLK, H), lambda i: (i, 0)),
        out_shape=jax.ShapeDtypeStruct((B, H), jnp.float32),
        scratch_shapes=[pltpu.VMEM((_KCAT, H), jnp.float32)],
    )(kid3, tv3, sid3, w1pad, b1pad, key_table, time_sig_table, W2, b2r,
      Wp, bpr)


# speedup vs baseline: 1.0736x; 1.0243x over previous
import jax
import jax.numpy as jnp
from jax import lax
from jax.experimental import pallas as pl
from jax.experimental.pallas import tpu as pltpu

_MIN_TEMPO = 90.0
_MAX_TEMPO = 140.0
_KCAT = 128


def _body(kid_ref, tv_ref, sid_ref, w1p_ref, b1p_ref, kt_ref, st_ref,
          w2_ref, b2_ref, wp_ref, bp_ref, out_ref, vcat_ref):
    blk = out_ref.shape[0]

    @pl.when(pl.program_id(0) == 0)
    def _fold():
        wp_key = wp_ref[0:64, :]
        wp_tmp = wp_ref[64:128, :]
        wp_sig = wp_ref[128:192, :]
        ktf = jnp.dot(kt_ref[...], wp_key, preferred_element_type=jnp.float32)
        stf = jnp.dot(st_ref[...], wp_sig, preferred_element_type=jnp.float32)
        w2f = jnp.dot(w2_ref[...], wp_tmp, preferred_element_type=jnp.float32)
        crow = jnp.dot(b2_ref[...], wp_tmp,
                       preferred_element_type=jnp.float32) + bp_ref[...]
        pad = jnp.zeros((_KCAT - 69, out_ref.shape[1]), jnp.float32)
        vcat_ref[...] = jnp.concatenate([ktf, stf, w2f, crow, pad], axis=0)

    kid = kid_ref[0, 0, :]
    sid = sid_ref[0, 0, :]
    tv = tv_ref[0, 0, :]

    lane = lax.broadcasted_iota(jnp.int32, (blk, _KCAT), 1)
    ohk = (kid[:, None] == lane).astype(jnp.float32)
    ohs = ((sid[:, None] + 26) == lane).astype(jnp.float32)
    tn = jnp.where(tv > 0, (tv - _MIN_TEMPO) / (_MAX_TEMPO - _MIN_TEMPO),
                   jnp.zeros_like(tv))
    h = jnp.maximum(tn[:, None] * w1p_ref[...] + b1p_ref[...], 0.0)
    m = ohk + ohs + h
    out_ref[...] = jnp.dot(m, vcat_ref[...],
                           preferred_element_type=jnp.float32)


def kernel(key_ids, tempo_values, time_sig_ids, key_table, time_sig_table,
           W1, b1, W2, b2, Wp, bp):
    B = key_ids.shape[0]
    H = Wp.shape[1]
    T = W1.shape[1]
    BLK = 4096 if B % 4096 == 0 else B
    NB = B // BLK

    def full_spec(shape):
        nd = len(shape)
        return pl.BlockSpec(shape, lambda i, _nd=nd: (0,) * _nd)

    b2r = b2.reshape(1, -1)
    bpr = bp.reshape(1, -1)
    w1pad = jnp.zeros((1, _KCAT), jnp.float32).at[0, 36:36 + T].set(W1[0])
    b1pad = (jnp.zeros((1, _KCAT), jnp.float32)
             .at[0, 36:36 + T].set(b1).at[0, 68].set(1.0))

    kid3 = key_ids.astype(jnp.int32).reshape(NB, 1, BLK)
    tv3 = tempo_values.reshape(NB, 1, BLK)
    sid3 = time_sig_ids.astype(jnp.int32).reshape(NB, 1, BLK)

    return pl.pallas_call(
        _body,
        grid=(NB,),
        in_specs=[
            pl.BlockSpec((1, 1, BLK), lambda i: (i, 0, 0)),
            pl.BlockSpec((1, 1, BLK), lambda i: (i, 0, 0)),
            pl.BlockSpec((1, 1, BLK), lambda i: (i, 0, 0)),
            full_spec((1, _KCAT)),
            full_spec((1, _KCAT)),
            full_spec(key_table.shape),
            full_spec(time_sig_table.shape),
            full_spec(W2.shape),
            full_spec(b2r.shape),
            full_spec(Wp.shape),
            full_spec(bpr.shape),
        ],
        out_specs=pl.BlockSpec((BLK, H), lambda i: (i, 0)),
        out_shape=jax.ShapeDtypeStruct((B, H), jnp.float32),
        scratch_shapes=[pltpu.VMEM((_KCAT, H), jnp.float32)],
    )(kid3, tv3, sid3, w1pad, b1pad, key_table, time_sig_table, W2, b2r,
      Wp, bpr)


# speedup vs baseline: 1.5637x; 1.4565x over previous
import jax
import jax.numpy as jnp
from jax import lax
from jax.experimental import pallas as pl

_MIN_TEMPO = 90.0
_MAX_TEMPO = 140.0


def _body(kid_ref, tv_ref, sid_ref, kt_ref, st_ref, w1_ref, b1_ref,
          w2_ref, b2_ref, wp_ref, bp_ref, out_ref):
    blk = out_ref.shape[0]
    nkey = kt_ref.shape[0]
    nsig = st_ref.shape[0]

    kid = kid_ref[0, 0, :]
    sid = sid_ref[0, 0, :]
    tv = tv_ref[0, 0, :]

    ohk = (kid[:, None] == lax.broadcasted_iota(jnp.int32, (blk, nkey), 1)
           ).astype(jnp.float32)
    ohs = (sid[:, None] == lax.broadcasted_iota(jnp.int32, (blk, nsig), 1)
           ).astype(jnp.float32)
    key_emb = jnp.dot(ohk, kt_ref[...], preferred_element_type=jnp.float32)
    sig_emb = jnp.dot(ohs, st_ref[...], preferred_element_type=jnp.float32)

    tn = jnp.where(tv > 0, (tv - _MIN_TEMPO) / (_MAX_TEMPO - _MIN_TEMPO),
                   jnp.zeros_like(tv))
    h = jnp.maximum(tn[:, None] * w1_ref[...] + b1_ref[...], 0.0)
    tempo_emb = jnp.dot(h, w2_ref[...],
                        preferred_element_type=jnp.float32) + b2_ref[...]

    combined = jnp.concatenate([key_emb, tempo_emb, sig_emb], axis=1)
    out_ref[...] = jnp.dot(combined, wp_ref[...],
                           preferred_element_type=jnp.float32) + bp_ref[...]


def kernel(key_ids, tempo_values, time_sig_ids, key_table, time_sig_table,
           W1, b1, W2, b2, Wp, bp):
    B = key_ids.shape[0]
    H = Wp.shape[1]
    BLK = 2048 if B % 2048 == 0 else B
    NB = B // BLK

    kid3 = key_ids.astype(jnp.int32).reshape(NB, 1, BLK)
    tv3 = tempo_values.reshape(NB, 1, BLK)
    sid3 = time_sig_ids.astype(jnp.int32).reshape(NB, 1, BLK)
    b1r = b1.reshape(1, -1)
    b2r = b2.reshape(1, -1)
    bpr = bp.reshape(1, -1)

    def blk_spec(shape):
        nd = len(shape)
        return pl.BlockSpec(shape, lambda i, _nd=nd: (0,) * _nd)

    return pl.pallas_call(
        _body,
        grid=(NB,),
        in_specs=[
            pl.BlockSpec((1, 1, BLK), lambda i: (i, 0, 0)),
            pl.BlockSpec((1, 1, BLK), lambda i: (i, 0, 0)),
            pl.BlockSpec((1, 1, BLK), lambda i: (i, 0, 0)),
            blk_spec(key_table.shape),
            blk_spec(time_sig_table.shape),
            blk_spec(W1.shape),
            blk_spec(b1r.shape),
            blk_spec(W2.shape),
            blk_spec(b2r.shape),
            blk_spec(Wp.shape),
            blk_spec(bpr.shape),
        ],
        out_specs=pl.BlockSpec((BLK, H), lambda i: (i, 0)),
        out_shape=jax.ShapeDtypeStruct((B, H), jnp.float32),
    )(kid3, tv3, sid3, key_table, time_sig_table, W1, b1r, W2, b2r, Wp, bpr)


# speedup vs baseline: 1.5977x; 1.0217x over previous
import jax
import jax.numpy as jnp
from jax import lax
from jax.experimental import pallas as pl

_MIN_TEMPO = 90.0
_MAX_TEMPO = 140.0


def _body(kid_ref, tv_ref, sid_ref, kt_ref, st_ref, w1_ref, b1_ref,
          w2_ref, b2_ref, wp_ref, bp_ref, out_ref):
    blk = out_ref.shape[0]
    nkey = kt_ref.shape[0]
    nsig = st_ref.shape[0]

    kid = kid_ref[0, 0, :]
    sid = sid_ref[0, 0, :]
    tv = tv_ref[0, 0, :]

    ohk = (kid[:, None] == lax.broadcasted_iota(jnp.int32, (blk, nkey), 1)
           ).astype(jnp.float32)
    ohs = (sid[:, None] == lax.broadcasted_iota(jnp.int32, (blk, nsig), 1)
           ).astype(jnp.float32)
    key_emb = jnp.dot(ohk, kt_ref[...], preferred_element_type=jnp.float32)
    sig_emb = jnp.dot(ohs, st_ref[...], preferred_element_type=jnp.float32)

    tn = jnp.where(tv > 0, (tv - _MIN_TEMPO) / (_MAX_TEMPO - _MIN_TEMPO),
                   jnp.zeros_like(tv))
    h = jnp.maximum(tn[:, None] * w1_ref[...] + b1_ref[...], 0.0)
    tempo_emb = jnp.dot(h, w2_ref[...],
                        preferred_element_type=jnp.float32) + b2_ref[...]

    combined = jnp.concatenate([key_emb, tempo_emb, sig_emb], axis=1)
    out_ref[...] = jnp.dot(combined, wp_ref[...],
                           preferred_element_type=jnp.float32) + bp_ref[...]


def kernel(key_ids, tempo_values, time_sig_ids, key_table, time_sig_table,
           W1, b1, W2, b2, Wp, bp):
    B = key_ids.shape[0]
    H = Wp.shape[1]
    BLK = 4096 if B % 4096 == 0 else B
    NB = B // BLK

    kid3 = key_ids.astype(jnp.int32).reshape(NB, 1, BLK)
    tv3 = tempo_values.reshape(NB, 1, BLK)
    sid3 = time_sig_ids.astype(jnp.int32).reshape(NB, 1, BLK)
    b1r = b1.reshape(1, -1)
    b2r = b2.reshape(1, -1)
    bpr = bp.reshape(1, -1)

    def blk_spec(shape):
        nd = len(shape)
        return pl.BlockSpec(shape, lambda i, _nd=nd: (0,) * _nd)

    return pl.pallas_call(
        _body,
        grid=(NB,),
        in_specs=[
            pl.BlockSpec((1, 1, BLK), lambda i: (i, 0, 0)),
            pl.BlockSpec((1, 1, BLK), lambda i: (i, 0, 0)),
            pl.BlockSpec((1, 1, BLK), lambda i: (i, 0, 0)),
            blk_spec(key_table.shape),
            blk_spec(time_sig_table.shape),
            blk_spec(W1.shape),
            blk_spec(b1r.shape),
            blk_spec(W2.shape),
            blk_spec(b2r.shape),
            blk_spec(Wp.shape),
            blk_spec(bpr.shape),
        ],
        out_specs=pl.BlockSpec((BLK, H), lambda i: (i, 0)),
        out_shape=jax.ShapeDtypeStruct((B, H), jnp.float32),
    )(kid3, tv3, sid3, key_table, time_sig_table, W1, b1r, W2, b2r, Wp, bpr)
